# Initial kernel scaffold; baseline (speedup 1.0000x reference)
import jax
import jax.numpy as jnp
from jax.experimental import pallas as pl


def kernel(llm_activations, We, be, Wd, bd, We1, be1, Wd1, bd1, We2, be2, Wd2, bd2):
    raise NotImplementedError("write your pallas kernel here")


# speedup vs baseline: 19.1169x; 19.1169x over previous
import functools

import jax
import jax.numpy as jnp
from jax.experimental import pallas as pl

_INT_MIN = -(2 ** 31)
_SEARCH_BITS = 31


def _key_to_float(key):
    u = jnp.where(key >= 0, key, jnp.int32(_INT_MIN) - key)
    return jax.lax.bitcast_convert_type(u, jnp.float32)


def _kth_threshold(v, k, check_sign=True):
    rows = v.shape[0]
    kf = jnp.float32(k)
    if check_sign:
        cnt0 = jnp.sum((v >= 0.0).astype(jnp.float32), axis=1, keepdims=True)
        cur = jnp.where(cnt0 >= kf, jnp.int32(0), jnp.int32(_INT_MIN))
        cur = jnp.broadcast_to(cur, (rows, 1)).astype(jnp.int32)
    else:
        cur = jnp.zeros((rows, 1), jnp.int32)
    for b in range(30, 30 - _SEARCH_BITS, -1):
        test = cur + jnp.int32(2 ** b)
        f = _key_to_float(test)
        cnt = jnp.sum((v >= f).astype(jnp.float32), axis=1, keepdims=True)
        cur = jnp.where(cnt >= kf, test, cur)
    return _key_to_float(cur)


_DN_NT = (((1,), (1,)), ((), ()))


def _encode_body(x_ref, we_ref, be_ref, we1_ref, be1_ref, we2_ref, be2_ref,
                 p_ref, p1_ref, p2_ref):
    x = x_ref[...]
    p_ref[...] = jax.lax.dot_general(
        x, we_ref[...], _DN_NT, preferred_element_type=jnp.float32) + be_ref[...]
    p1_ref[...] = jax.lax.dot_general(
        x, we1_ref[...], _DN_NT, preferred_element_type=jnp.float32) + be1_ref[...]
    p2_ref[...] = jax.lax.dot_general(
        x, we2_ref[...], _DN_NT, preferred_element_type=jnp.float32) + be2_ref[...]


def _select_body(k0, k1, p_ref, p1_ref, p2_ref, s_ref, f1_ref, f2_ref):
    pre = p_ref[...]
    t0 = _kth_threshold(pre, k0, check_sign=True)
    mask = pre >= t0
    s_ref[...] = jnp.where(mask, pre, 0.0)
    maskc = mask & (pre != 0.0)

    m1 = jnp.where(maskc, p1_ref[...], 0.0)
    t1 = _kth_threshold(m1, k1, check_sign=False)
    f1_ref[...] = jnp.where(m1 >= t1, m1, 0.0)

    m2 = jnp.where(maskc, p2_ref[...], 0.0)
    t2 = _kth_threshold(m2, k1, check_sign=False)
    f2_ref[...] = jnp.where(m2 >= t2, m2, 0.0)


def _decode_body(s_ref, f1_ref, f2_ref, wd_ref, wd1_ref, wd2_ref, bsum_ref,
                 out_ref, *, grid_h):
    h = pl.program_id(0)
    part = jax.lax.dot_general(
        s_ref[...], wd_ref[...], _DN_NT, preferred_element_type=jnp.float32)
    part += jax.lax.dot_general(
        f1_ref[...], wd1_ref[...], _DN_NT, preferred_element_type=jnp.float32)
    part += jax.lax.dot_general(
        f2_ref[...], wd2_ref[...], _DN_NT, preferred_element_type=jnp.float32)

    @pl.when(h == 0)
    def _():
        out_ref[...] = part + bsum_ref[...]

    @pl.when(h != 0)
    def _():
        out_ref[...] += part


def kernel(llm_activations, We, be, Wd, bd, We1, be1, Wd1, bd1, We2, be2,
           Wd2, bd2):
    b, seq, llm_h = llm_activations.shape
    sae_h = We.shape[0]
    k0, k1 = 150, 75

    x = llm_activations.reshape(seq, llm_h)
    h_tile = min(512, sae_h)
    grid_h = sae_h // h_tile
    r_sel = min(32, seq)
    grid_sel = seq // r_sel

    be_r = be.reshape(1, sae_h)
    be1_r = be1.reshape(1, sae_h)
    be2_r = be2.reshape(1, sae_h)
    bsum = (bd + bd1 + bd2).reshape(1, llm_h)

    f32 = jnp.float32
    pre_shape = jax.ShapeDtypeStruct((seq, sae_h), f32)

    w_spec = pl.BlockSpec((h_tile, llm_h), lambda h: (h, 0))
    bias_spec = pl.BlockSpec((1, h_tile), lambda h: (0, h))
    penc_spec = pl.BlockSpec((seq, h_tile), lambda h: (0, h))
    pre, pre1, pre2 = pl.pallas_call(
        _encode_body,
        grid=(grid_h,),
        in_specs=[
            pl.BlockSpec((seq, llm_h), lambda h: (0, 0)),
            w_spec, bias_spec, w_spec, bias_spec, w_spec, bias_spec,
        ],
        out_specs=[penc_spec, penc_spec, penc_spec],
        out_shape=[pre_shape, pre_shape, pre_shape],
    )(x, We, be_r, We1, be1_r, We2, be2_r)

    sel_spec = pl.BlockSpec((r_sel, sae_h), lambda t: (t, 0))
    sae, f1, f2 = pl.pallas_call(
        functools.partial(_select_body, k0, k1),
        grid=(grid_sel,),
        in_specs=[sel_spec, sel_spec, sel_spec],
        out_specs=[sel_spec, sel_spec, sel_spec],
        out_shape=[pre_shape, pre_shape, pre_shape],
    )(pre, pre1, pre2)

    act_spec = pl.BlockSpec((seq, h_tile), lambda h: (0, h))
    wd_spec = pl.BlockSpec((llm_h, h_tile), lambda h: (0, h))
    out = pl.pallas_call(
        functools.partial(_decode_body, grid_h=grid_h),
        grid=(grid_h,),
        in_specs=[
            act_spec, act_spec, act_spec,
            wd_spec, wd_spec, wd_spec,
            pl.BlockSpec((1, llm_h), lambda h: (0, 0)),
        ],
        out_specs=pl.BlockSpec((seq, llm_h), lambda h: (0, 0)),
        out_shape=jax.ShapeDtypeStruct((seq, llm_h), f32),
    )(sae, f1, f2, Wd, Wd1, Wd2, bsum)

    return out.reshape(b, seq, llm_h)
